# baseline (speedup 1.0000x reference)
import jax
import jax.numpy as jnp
from jax import lax
from jax.experimental import pallas as pl
from jax.experimental.pallas import tpu as pltpu
from jax.experimental.pallas import tpu_sc as plsc

B, H, W, C = 4, 112, 112, 96
HW = H * W
OHW = 4 * HW
CW = 16
NQ = C // CW
NWIN = B * NQ
NSUB = 16
WIN_PER_CORE = NWIN // 2
RPS = HW // NSUB
EPS = RPS * CW
CHUNK = 128
NCH = EPS // CHUNK
ACC = OHW * CW
ZSPAN = ACC // NSUB
ZB = 3136


def _sc_body(upd_hbm, idx_hbm, out_hbm, acc, vstage, rawi, dsts0, vals0,
             dsts1, vals1, zbuf, dsem, lsem, fsem, zsem):
    cid = lax.axis_index("c")
    sid = lax.axis_index("s")
    lanes = lax.iota(jnp.int32, 16)
    bufs = ((dsts0, vals0), (dsts1, vals1))

    @pl.loop(0, ZB // 16)
    def _(i):
        zbuf[pl.ds(i * 16, 16)] = jnp.zeros((16,), jnp.float32)

    def load(t):
        w = cid * WIN_PER_CORE + t
        b = w // NQ
        c0 = (w % NQ) * CW
        r0 = sid * RPS
        pltpu.async_copy(idx_hbm.at[b, pl.ds(r0, RPS), pl.ds(c0, CW)], rawi,
                         lsem)
        pltpu.async_copy(upd_hbm.at[b, pl.ds(r0, RPS), pl.ds(c0, CW)], vstage,
                         lsem)

    def load_wait(t):
        w = cid * WIN_PER_CORE + t
        b = w // NQ
        c0 = (w % NQ) * CW
        r0 = sid * RPS
        pltpu.make_async_copy(
            idx_hbm.at[b, pl.ds(r0, RPS), pl.ds(c0, CW)], rawi, lsem).wait()
        pltpu.make_async_copy(
            upd_hbm.at[b, pl.ds(r0, RPS), pl.ds(c0, CW)], vstage, lsem).wait()

    def decode(dsts, vals):
        @pl.loop(0, RPS, unroll=4)
        def _(r):
            rv = rawi[r, pl.ds(0, CW)]
            y = lax.shift_right_logical(rv, 5).astype(jnp.float32)
            p = (y * jnp.float32(1.0 / 3.0)).astype(jnp.int32)
            dsts[pl.ds(r * CW, CW)] = p * CW + lanes
            vals[pl.ds(r * CW, CW)] = vstage[r, pl.ds(0, CW)]

    load(0)
    load_wait(0)
    decode(dsts0, vals0)

    for t in range(WIN_PER_CORE):
        w = cid * WIN_PER_CORE + t
        z0 = sid * ZSPAN
        dsts, vals = bufs[t % 2]
        ndsts, nvals = bufs[(t + 1) % 2]

        if t > 0:
            pltpu.make_async_copy(
                acc.at[pl.ds(z0, ZSPAN)],
                out_hbm.at[w - 1, pl.ds(z0, ZSPAN)], fsem).wait()

        @pl.loop(0, ZSPAN // ZB)
        def _(z):
            pltpu.async_copy(zbuf, acc.at[pl.ds(z0 + z * ZB, ZB)], zsem)

        @pl.loop(0, ZSPAN // ZB)
        def _(z):
            pltpu.make_async_copy(
                zbuf, acc.at[pl.ds(z0 + z * ZB, ZB)], zsem).wait()
        plsc.subcore_barrier()

        @pl.loop(0, NCH)
        def _(ci):
            pltpu.async_copy(vals.at[pl.ds(ci * CHUNK, CHUNK)],
                             acc.at[dsts.at[pl.ds(ci * CHUNK, CHUNK)]],
                             dsem, add=True)

        if t + 1 < WIN_PER_CORE:
            load(t + 1)
            load_wait(t + 1)
            decode(ndsts, nvals)

        @pl.loop(0, NCH)
        def _(ci):
            pltpu.make_async_copy(
                vals.at[pl.ds(ci * CHUNK, CHUNK)],
                acc.at[dsts.at[pl.ds(ci * CHUNK, CHUNK)]], dsem).wait()
        plsc.subcore_barrier()

        pltpu.async_copy(acc.at[pl.ds(z0, ZSPAN)],
                         out_hbm.at[w, pl.ds(z0, ZSPAN)], fsem)

    z0 = sid * ZSPAN
    pltpu.make_async_copy(
        acc.at[pl.ds(z0, ZSPAN)],
        out_hbm.at[cid * WIN_PER_CORE + WIN_PER_CORE - 1, pl.ds(z0, ZSPAN)],
        fsem).wait()


_sc_call = pl.kernel(
    _sc_body,
    out_type=jax.ShapeDtypeStruct((NWIN, ACC), jnp.float32),
    mesh=plsc.VectorSubcoreMesh(core_axis_name="c", subcore_axis_name="s"),
    scratch_types=[
        pltpu.VMEM_SHARED((ACC,), jnp.float32),
        pltpu.VMEM((RPS, CW), jnp.float32),
        pltpu.VMEM((RPS, CW), jnp.int32),
        pltpu.VMEM((EPS,), jnp.int32),
        pltpu.VMEM((EPS,), jnp.float32),
        pltpu.VMEM((EPS,), jnp.int32),
        pltpu.VMEM((EPS,), jnp.float32),
        pltpu.VMEM((ZB,), jnp.float32),
        pltpu.SemaphoreType.DMA,
        pltpu.SemaphoreType.DMA,
        pltpu.SemaphoreType.DMA,
        pltpu.SemaphoreType.DMA,
    ],
    compiler_params=pltpu.CompilerParams(
        use_tc_tiling_on_sc=False, needs_layout_passes=False),
)

FR = OHW // NSUB


def _relayout_body(win_hbm, out_hbm, bounce):
    cid = lax.axis_index("c")
    sid = lax.axis_index("s")

    @pl.loop(0, WIN_PER_CORE)
    def _(t):
        w = cid * WIN_PER_CORE + t
        b = w // NQ
        c0 = (w % NQ) * CW
        r0 = sid * FR
        pltpu.sync_copy(win_hbm.at[w, pl.ds(r0, FR), :], bounce)
        pltpu.sync_copy(bounce,
                        out_hbm.at[b, pl.ds(r0, FR), pl.ds(c0, CW)])


_relayout = pl.kernel(
    _relayout_body,
    out_type=jax.ShapeDtypeStruct((B, OHW, C), jnp.float32),
    mesh=plsc.VectorSubcoreMesh(core_axis_name="c", subcore_axis_name="s"),
    scratch_types=[pltpu.VMEM((FR, CW), jnp.float32)],
    compiler_params=pltpu.CompilerParams(
        use_tc_tiling_on_sc=False, needs_layout_passes=False),
)


@jax.jit
def kernel(updates, indices):
    upd = updates.reshape(B, HW, C)
    idx = indices.astype(jnp.int32).reshape(B, HW, C)
    win = _sc_call(upd, idx)
    out = _relayout(win.reshape(NWIN, OHW, CW))
    return out.reshape(B, 2 * H, 2 * W, C)


# speedup vs baseline: 1.0734x; 1.0734x over previous
import jax
import jax.numpy as jnp
from jax import lax
from jax.experimental import pallas as pl
from jax.experimental.pallas import tpu as pltpu
from jax.experimental.pallas import tpu_sc as plsc

B, H, W, C = 4, 112, 112, 96
HW = H * W
OHW = 4 * HW
CW = 16
NQ = C // CW
NWIN = B * NQ
NSUB = 16
WIN_PER_CORE = NWIN // 2
RPS = HW // NSUB
EPS = RPS * CW
CHUNK = 128
NCH = EPS // CHUNK
ACC = OHW * CW
ZSPAN = ACC // NSUB
ZB = 6272


def _sc_body(upd_hbm, idx_hbm, out_hbm, acc, vstage, rawi, dsts, vals, zbuf,
             dsem, lsem, fsem, zsem):
    cid = lax.axis_index("c")
    sid = lax.axis_index("s")
    lanes = lax.iota(jnp.int32, 16)

    @pl.loop(0, ZB // 16)
    def _(i):
        zbuf[pl.ds(i * 16, 16)] = jnp.zeros((16,), jnp.float32)

    def load(t):
        w = cid * WIN_PER_CORE + t
        b = w // NQ
        c0 = (w % NQ) * CW
        r0 = sid * RPS
        pltpu.async_copy(idx_hbm.at[b, pl.ds(r0, RPS), pl.ds(c0, CW)], rawi,
                         lsem)
        pltpu.async_copy(upd_hbm.at[b, pl.ds(r0, RPS), pl.ds(c0, CW)], vstage,
                         lsem)

    load(0)

    @pl.loop(0, WIN_PER_CORE)
    def _(t):
        w = cid * WIN_PER_CORE + t
        z0 = sid * ZSPAN

        @pl.when(t > 0)
        def _():
            pltpu.make_async_copy(
                acc.at[pl.ds(z0, ZSPAN)],
                out_hbm.at[w - 1, pl.ds(z0, ZSPAN)], fsem).wait()

        @pl.loop(0, ZSPAN // ZB)
        def _(z):
            pltpu.async_copy(zbuf, acc.at[pl.ds(z0 + z * ZB, ZB)], zsem)

        @pl.loop(0, ZSPAN // ZB)
        def _(z):
            pltpu.make_async_copy(
                zbuf, acc.at[pl.ds(z0 + z * ZB, ZB)], zsem).wait()
        plsc.subcore_barrier()

        r0 = sid * RPS
        b = w // NQ
        c0 = (w % NQ) * CW
        pltpu.make_async_copy(
            idx_hbm.at[b, pl.ds(r0, RPS), pl.ds(c0, CW)], rawi, lsem).wait()
        pltpu.make_async_copy(
            upd_hbm.at[b, pl.ds(r0, RPS), pl.ds(c0, CW)], vstage, lsem).wait()

        @pl.loop(0, RPS, unroll=4)
        def _(r):
            rv = rawi[r, pl.ds(0, CW)]
            y = lax.shift_right_logical(rv, 5).astype(jnp.float32)
            p = (y * jnp.float32(1.0 / 3.0)).astype(jnp.int32)
            dsts[pl.ds(r * CW, CW)] = p * CW + lanes
            vals[pl.ds(r * CW, CW)] = vstage[r, pl.ds(0, CW)]

        @pl.loop(0, NCH)
        def _(ci):
            pltpu.async_copy(vals.at[pl.ds(ci * CHUNK, CHUNK)],
                             acc.at[dsts.at[pl.ds(ci * CHUNK, CHUNK)]],
                             dsem, add=True)

        @pl.loop(0, NCH)
        def _(ci):
            pltpu.make_async_copy(
                vals.at[pl.ds(ci * CHUNK, CHUNK)],
                acc.at[dsts.at[pl.ds(ci * CHUNK, CHUNK)]], dsem).wait()
        plsc.subcore_barrier()

        pltpu.async_copy(acc.at[pl.ds(z0, ZSPAN)],
                         out_hbm.at[w, pl.ds(z0, ZSPAN)], fsem)

        @pl.when(t + 1 < WIN_PER_CORE)
        def _():
            load(t + 1)

    z0 = sid * ZSPAN
    pltpu.make_async_copy(
        acc.at[pl.ds(z0, ZSPAN)],
        out_hbm.at[cid * WIN_PER_CORE + WIN_PER_CORE - 1, pl.ds(z0, ZSPAN)],
        fsem).wait()


_sc_call = pl.kernel(
    _sc_body,
    out_type=jax.ShapeDtypeStruct((NWIN, ACC), jnp.float32),
    mesh=plsc.VectorSubcoreMesh(core_axis_name="c", subcore_axis_name="s"),
    scratch_types=[
        pltpu.VMEM_SHARED((ACC,), jnp.float32),
        pltpu.VMEM((RPS, CW), jnp.float32),
        pltpu.VMEM((RPS, CW), jnp.int32),
        pltpu.VMEM((EPS,), jnp.int32),
        pltpu.VMEM((EPS,), jnp.float32),
        pltpu.VMEM((ZB,), jnp.float32),
        pltpu.SemaphoreType.DMA,
        pltpu.SemaphoreType.DMA,
        pltpu.SemaphoreType.DMA,
        pltpu.SemaphoreType.DMA,
    ],
    compiler_params=pltpu.CompilerParams(
        use_tc_tiling_on_sc=False, needs_layout_passes=False),
)

FR = OHW // NSUB


def _relayout_body(win_hbm, out_hbm, bounce0, bounce1, rsem, wsem):
    cid = lax.axis_index("c")
    sid = lax.axis_index("s")
    r0 = sid * FR
    bufs = (bounce0, bounce1)

    def rd(t):
        w = cid * WIN_PER_CORE + t
        pltpu.async_copy(win_hbm.at[w, pl.ds(r0, FR), :], bufs[t % 2], rsem)

    def rd_wait(t):
        w = cid * WIN_PER_CORE + t
        pltpu.make_async_copy(
            win_hbm.at[w, pl.ds(r0, FR), :], bufs[t % 2], rsem).wait()

    def wr(t):
        w = cid * WIN_PER_CORE + t
        b = w // NQ
        c0 = (w % NQ) * CW
        return pltpu.make_async_copy(
            bufs[t % 2], out_hbm.at[b, pl.ds(r0, FR), pl.ds(c0, CW)], wsem)

    rd(0)
    for t in range(WIN_PER_CORE):
        rd_wait(t)
        pltpu.async_copy(
            bufs[t % 2],
            out_hbm.at[(cid * WIN_PER_CORE + t) // NQ, pl.ds(r0, FR),
                       pl.ds(((cid * WIN_PER_CORE + t) % NQ) * CW, CW)],
            wsem)
        if t + 1 < WIN_PER_CORE:
            rd(t + 1)
        if t > 0:
            wr(t - 1).wait()
    wr(WIN_PER_CORE - 1).wait()


_relayout = pl.kernel(
    _relayout_body,
    out_type=jax.ShapeDtypeStruct((B, OHW, C), jnp.float32),
    mesh=plsc.VectorSubcoreMesh(core_axis_name="c", subcore_axis_name="s"),
    scratch_types=[
        pltpu.VMEM((FR, CW), jnp.float32),
        pltpu.VMEM((FR, CW), jnp.float32),
        pltpu.SemaphoreType.DMA,
        pltpu.SemaphoreType.DMA,
    ],
    compiler_params=pltpu.CompilerParams(
        use_tc_tiling_on_sc=False, needs_layout_passes=False),
)


@jax.jit
def kernel(updates, indices):
    upd = updates.reshape(B, HW, C)
    idx = indices.astype(jnp.int32).reshape(B, HW, C)
    win = _sc_call(upd, idx)
    out = _relayout(win.reshape(NWIN, OHW, CW))
    return out.reshape(B, 2 * H, 2 * W, C)
